# baseline (speedup 1.0000x reference)
import functools

import jax
import jax.numpy as jnp
from jax import lax
from jax.experimental import pallas as pl
from jax.experimental.pallas import tpu as pltpu
from jax.experimental.pallas import tpu_sc as plsc

CH = 32
NBUF = 2


def _pack_sc(xf, B, L, d):
    info = plsc.get_sparse_core_info()
    NC, NS, NL = info.num_cores, info.num_subcores, info.num_lanes
    NW = NC * NS
    R = (L * B) // NW
    n_chunks = R // CH
    n_groups = n_chunks // NBUF

    mesh = plsc.VectorSubcoreMesh(core_axis_name="c", subcore_axis_name="s")

    @functools.partial(
        pl.kernel,
        mesh=mesh,
        out_type=jax.ShapeDtypeStruct((L * B, d), jnp.float32),
        scratch_types=[
            pltpu.VMEM((R,), jnp.int32),
        ]
        + [pltpu.VMEM((CH, d), jnp.float32)] * NBUF
        + [pltpu.SemaphoreType.DMA] * (2 * NBUF),
    )
    def k(x_hbm, out_hbm, idxv, *bufsem):
        bufs, rsems, wsems = (
            bufsem[:NBUF], bufsem[NBUF:2 * NBUF], bufsem[2 * NBUF:])
        wid = lax.axis_index("s") * NC + lax.axis_index("c")
        base = wid * R
        tbase = base // B

        def gen(j, _):
            idxv[pl.ds(NL * j, NL)] = lax.iota(jnp.int32, NL) * L + (tbase + j)
            return _

        lax.fori_loop(0, R // NL, gen, None)

        def gather(c, s):
            return pltpu.make_async_copy(
                x_hbm.at[idxv.at[pl.ds(c * CH, CH)]], bufs[s], rsems[s])

        def put(c, s):
            return pltpu.make_async_copy(
                bufs[s], out_hbm.at[pl.ds(base + c * CH, CH)], wsems[s])

        for s in range(NBUF):
            gather(s, s).start()

        def step(j, _):
            c0 = j * NBUF
            for s in range(NBUF):
                gather(c0 + s, s).wait()
                put(c0 + s, s).start()
            for s in range(NBUF):
                put(c0 + s, s).wait()
                gather(c0 + NBUF + s, s).start()
            return _

        lax.fori_loop(0, n_groups - 1, step, None)

        cl = (n_groups - 1) * NBUF
        for s in range(NBUF):
            gather(cl + s, s).wait()
            put(cl + s, s).start()
        for s in range(NBUF):
            put(cl + s, s).wait()

    return k(xf)


def kernel(x):
    B, L, d = x.shape
    data = _pack_sc(x.reshape(B * L, d), B, L, d)
    batch_sizes = jnp.full((L,), B, dtype=jnp.int64)
    sorted_indices = jnp.arange(B, dtype=jnp.int64)
    unsorted_indices = jnp.arange(B, dtype=jnp.int64)
    return data, batch_sizes, sorted_indices, unsorted_indices


# speedup vs baseline: 1.0284x; 1.0284x over previous
import functools

import jax
import jax.numpy as jnp
from jax import lax
from jax.experimental import pallas as pl
from jax.experimental.pallas import tpu as pltpu
from jax.experimental.pallas import tpu_sc as plsc

CH = 16
NBUF = 4


def _pack_sc(xf, B, L, d):
    info = plsc.get_sparse_core_info()
    NC, NS, NL = info.num_cores, info.num_subcores, info.num_lanes
    NW = NC * NS
    R = (L * B) // NW
    n_chunks = R // CH
    n_groups = n_chunks // NBUF

    mesh = plsc.VectorSubcoreMesh(core_axis_name="c", subcore_axis_name="s")

    @functools.partial(
        pl.kernel,
        mesh=mesh,
        out_type=jax.ShapeDtypeStruct((L * B, d), jnp.float32),
        scratch_types=[
            pltpu.VMEM((R,), jnp.int32),
        ]
        + [pltpu.VMEM((CH, d), jnp.float32)] * NBUF
        + [pltpu.SemaphoreType.DMA] * (2 * NBUF),
    )
    def k(x_hbm, out_hbm, idxv, *bufsem):
        bufs, rsems, wsems = (
            bufsem[:NBUF], bufsem[NBUF:2 * NBUF], bufsem[2 * NBUF:])
        wid = lax.axis_index("s") * NC + lax.axis_index("c")
        base = wid * R
        tbase = base // B

        def gen(j, _):
            idxv[pl.ds(NL * j, NL)] = lax.iota(jnp.int32, NL) * L + (tbase + j)
            return _

        lax.fori_loop(0, R // NL, gen, None)

        def gather(c, s):
            return pltpu.make_async_copy(
                x_hbm.at[idxv.at[pl.ds(c * CH, CH)]], bufs[s], rsems[s])

        def put(c, s):
            return pltpu.make_async_copy(
                bufs[s], out_hbm.at[pl.ds(base + c * CH, CH)], wsems[s])

        for s in range(NBUF):
            gather(s, s).start()

        def step(j, _):
            c0 = j * NBUF
            for s in range(NBUF):
                gather(c0 + s, s).wait()
                put(c0 + s, s).start()
            for s in range(NBUF):
                put(c0 + s, s).wait()
                gather(c0 + NBUF + s, s).start()
            return _

        lax.fori_loop(0, n_groups - 1, step, None)

        cl = (n_groups - 1) * NBUF
        for s in range(NBUF):
            gather(cl + s, s).wait()
            put(cl + s, s).start()
        for s in range(NBUF):
            put(cl + s, s).wait()

    return k(xf)


def kernel(x):
    B, L, d = x.shape
    data = _pack_sc(x.reshape(B * L, d), B, L, d)
    batch_sizes = jnp.full((L,), B, dtype=jnp.int64)
    sorted_indices = jnp.arange(B, dtype=jnp.int64)
    unsorted_indices = jnp.arange(B, dtype=jnp.int64)
    return data, batch_sizes, sorted_indices, unsorted_indices


# speedup vs baseline: 1.0391x; 1.0104x over previous
import functools

import jax
import jax.numpy as jnp
from jax import lax
from jax.experimental import pallas as pl
from jax.experimental.pallas import tpu as pltpu
from jax.experimental.pallas import tpu_sc as plsc

CH = 16
NBUF = 4


def _pack_sc(xf, B, L, d):
    info = plsc.get_sparse_core_info()
    NC, NS, NL = info.num_cores, info.num_subcores, info.num_lanes
    NW = NC * NS
    R = (L * B) // NW
    n_chunks = R // CH
    n_groups = n_chunks // NBUF

    mesh = plsc.VectorSubcoreMesh(core_axis_name="c", subcore_axis_name="s")

    @functools.partial(
        pl.kernel,
        mesh=mesh,
        out_type=jax.ShapeDtypeStruct((L * B, d), jnp.float32),
        scratch_types=[
            pltpu.VMEM((n_chunks, CH), jnp.int32),
        ]
        + [pltpu.VMEM((CH, d), jnp.float32)] * NBUF
        + [pltpu.SemaphoreType.DMA] * (2 * NBUF),
    )
    def k(x_hbm, out_hbm, idxv, *bufsem):
        bufs, rsems, wsems = (
            bufsem[:NBUF], bufsem[NBUF:2 * NBUF], bufsem[2 * NBUF:])
        wid = lax.axis_index("s") * NC + lax.axis_index("c")
        base = wid * R
        b = base // L
        t0 = base % L

        def gen(c, _):
            idxv[c, :] = (t0 + CH * c + lax.iota(jnp.int32, NL)) * B + b
            return _

        lax.fori_loop(0, n_chunks, gen, None)

        def get(c, s):
            return pltpu.make_async_copy(
                x_hbm.at[pl.ds(base + c * CH, CH)], bufs[s], rsems[s])

        def put(c, s):
            return pltpu.make_async_copy(
                bufs[s], out_hbm.at[idxv.at[c]], wsems[s])

        for s in range(NBUF):
            get(s, s).start()

        def step(j, _):
            c0 = j * NBUF
            for s in range(NBUF):
                get(c0 + s, s).wait()
                put(c0 + s, s).start()
            for s in range(NBUF):
                put(c0 + s, s).wait()
                get(c0 + NBUF + s, s).start()
            return _

        lax.fori_loop(0, n_groups - 1, step, None)

        cl = (n_groups - 1) * NBUF
        for s in range(NBUF):
            get(cl + s, s).wait()
            put(cl + s, s).start()
        for s in range(NBUF):
            put(cl + s, s).wait()

    return k(xf)


def kernel(x):
    B, L, d = x.shape
    data = _pack_sc(x.reshape(B * L, d), B, L, d)
    batch_sizes = jnp.full((L,), B, dtype=jnp.int64)
    sorted_indices = jnp.arange(B, dtype=jnp.int64)
    unsorted_indices = jnp.arange(B, dtype=jnp.int64)
    return data, batch_sizes, sorted_indices, unsorted_indices
